# Initial kernel scaffold; baseline (speedup 1.0000x reference)
import jax
import jax.numpy as jnp
from jax.experimental import pallas as pl


def kernel(x, edge_index, edge_type, W1, root1, b1, W2, root2, b2):
    raise NotImplementedError("write your pallas kernel here")


# speedup vs baseline: 10.8770x; 10.8770x over previous
import functools
import jax
import jax.numpy as jnp
from jax import lax
from jax.experimental import pallas as pl
from jax.experimental.pallas import tpu as pltpu
from jax.experimental.pallas import tpu_sc as plsc

N = 10000
E = 320000
D = 128
R = 8

NC = 2
NS = 16
NW = NC * NS

B = 128
EPW = 10112
NB = EPW // B
E_PAD = NW * EPW

S_PAD = 81920
SEG_DUMP = N * R
CPS = S_PAD // NS

AGG_N = 10240
APS = AGG_N // NS

_mesh = plsc.VectorSubcoreMesh(core_axis_name="c", subcore_axis_name="s")


def _fill1d(ref, n, val):
  def body(i, carry):
    ref[pl.ds(i * 16, 16)] = jnp.full((16,), val, dtype=ref.dtype)
    return carry
  lax.fori_loop(0, n // 16, body, 0)


def _zero_rows(ref):
  def body(i, carry):
    e = i // (D // 16)
    k = i % (D // 16)
    ref[e, pl.ds(k * 16, 16)] = jnp.zeros((16,), dtype=ref.dtype)
    return carry
  lax.fori_loop(0, B * (D // 16), body, 0)


@functools.partial(
    pl.kernel,
    out_type=jax.ShapeDtypeStruct((NC, S_PAD), jnp.float32),
    mesh=_mesh,
    scratch_types=[
        pltpu.VMEM((B,), jnp.int32),
        pltpu.VMEM((B,), jnp.float32),
        pltpu.VMEM((B,), jnp.float32),
        pltpu.VMEM_SHARED((S_PAD,), jnp.float32),
    ],
    compiler_params=pltpu.CompilerParams(needs_layout_passes=False),
)
def _sc_counts(seg_hbm, cnt_hbm, seg_v, ones_v, zero_v, cnt_sh):
  c = lax.axis_index("c")
  s = lax.axis_index("s")
  w = c * NS + s

  _fill1d(zero_v, B, 0.0)
  _fill1d(ones_v, B, 1.0)

  def zero_body(k, carry):
    pltpu.sync_copy(zero_v, cnt_sh.at[pl.ds(s * CPS + k * B, B)])
    return carry
  lax.fori_loop(0, CPS // B, zero_body, 0)
  plsc.subcore_barrier()

  def acc_body(b, carry):
    pltpu.sync_copy(seg_hbm.at[pl.ds(w * EPW + b * B, B)], seg_v)
    pltpu.sync_copy(ones_v, cnt_sh.at[seg_v], add=True)
    return carry
  lax.fori_loop(0, NB, acc_body, 0)
  plsc.subcore_barrier()

  pltpu.sync_copy(cnt_sh.at[pl.ds(s * CPS, CPS)], cnt_hbm.at[c, pl.ds(s * CPS, CPS)])


@functools.partial(
    pl.kernel,
    out_type=jax.ShapeDtypeStruct((NC, AGG_N, D), jnp.float32),
    mesh=_mesh,
    scratch_types=[
        pltpu.VMEM((B,), jnp.int32),
        pltpu.VMEM((B,), jnp.int32),
        pltpu.VMEM((B,), jnp.int32),
        pltpu.VMEM((B,), jnp.float32),
        pltpu.VMEM((B, D), jnp.float32),
        pltpu.SemaphoreType.DMA,
        pltpu.SemaphoreType.DMA,
        pltpu.VMEM_SHARED((AGG_N, D), jnp.float32),
    ],
    compiler_params=pltpu.CompilerParams(needs_layout_passes=False),
)
def _sc_scatter(xw_hbm, widx_hbm, dst_hbm, seg_hbm, inv_hbm, agg_hbm,
                widx_v, dst_v, seg_v, sc_v, rows_v, sem, sem2, agg_sh):
  c = lax.axis_index("c")
  s = lax.axis_index("s")
  w = c * NS + s

  _zero_rows(rows_v)
  def zero_body(k, carry):
    pltpu.sync_copy(rows_v, agg_sh.at[pl.ds(s * APS + k * B, B)])
    return carry
  lax.fori_loop(0, APS // B, zero_body, 0)

  plsc.subcore_barrier()

  def edge_batch(b, carry):
    base = w * EPW + b * B
    pltpu.sync_copy(widx_hbm.at[pl.ds(base, B)], widx_v)
    pltpu.sync_copy(seg_hbm.at[pl.ds(base, B)], seg_v)
    pltpu.sync_copy(dst_hbm.at[pl.ds(base, B)], dst_v)
    row_cp = pltpu.async_copy(xw_hbm.at[widx_v], rows_v, sem)
    pltpu.async_copy(inv_hbm.at[seg_v], sc_v, sem2).wait()
    row_cp.wait()

    def one_edge(e, carry2):
      spl = plsc.load_gather(sc_v, [jnp.full((16,), e, jnp.int32)])
      for k in range(D // 16):
        rows_v[e, pl.ds(k * 16, 16)] = rows_v[e, pl.ds(k * 16, 16)] * spl
      return carry2
    lax.fori_loop(0, B, one_edge, 0, unroll=4)

    pltpu.sync_copy(rows_v, agg_sh.at[dst_v], add=True)
    return carry
  lax.fori_loop(0, NB, edge_batch, 0)
  plsc.subcore_barrier()

  def dump_body(k, carry):
    pltpu.sync_copy(agg_sh.at[pl.ds(s * APS + k * B, B)],
                    agg_hbm.at[c, pl.ds(s * APS + k * B, B)])
    return carry
  lax.fori_loop(0, APS // B, dump_body, 0)


def _tc_xw_body(x_ref, w_ref, out_ref):
  out_ref[0] = jnp.dot(x_ref[...], w_ref[0],
                       preferred_element_type=jnp.float32)


def _tc_xw(x, W):
  return pl.pallas_call(
      _tc_xw_body,
      grid=(R,),
      in_specs=[
          pl.BlockSpec((N, D), lambda r: (0, 0)),
          pl.BlockSpec((1, D, D), lambda r: (r, 0, 0)),
      ],
      out_specs=pl.BlockSpec((1, N, D), lambda r: (r, 0, 0)),
      out_shape=jax.ShapeDtypeStruct((R, N, D), jnp.float32),
  )(x, W)


def _tc_inv_body(cnt_ref, inv_ref):
  c = cnt_ref[0] + cnt_ref[1]
  inv_ref[...] = 1.0 / jnp.maximum(c, 1.0)


def _tc_inv(counts):
  return pl.pallas_call(
      _tc_inv_body,
      out_shape=jax.ShapeDtypeStruct((S_PAD // 128, 128), jnp.float32),
  )(counts.reshape(2, S_PAD // 128, 128)).reshape(S_PAD)


def _tc_combine_body(agg_ref, x_ref, root_ref, b_ref, out_ref, *, relu):
  acc = agg_ref[0, :N, :] + agg_ref[1, :N, :]
  acc = acc + jnp.dot(x_ref[...], root_ref[...],
                      preferred_element_type=jnp.float32) + b_ref[...]
  out_ref[...] = jnp.maximum(acc, 0.0) if relu else acc


def _tc_combine(agg, x, root, b, relu):
  return pl.pallas_call(
      functools.partial(_tc_combine_body, relu=relu),
      out_shape=jax.ShapeDtypeStruct((N, D), jnp.float32),
  )(agg, x, root, b.reshape(1, D))


def kernel(x, edge_index, edge_type, W1, root1, b1, W2, root2, b2):
  src = edge_index[0].astype(jnp.int32)
  dst = edge_index[1].astype(jnp.int32)
  et = edge_type.astype(jnp.int32)

  pad = E_PAD - E
  widx_p = jnp.concatenate([et * N + src, jnp.zeros((pad,), jnp.int32)])
  seg_p = jnp.concatenate([dst * R + et, jnp.full((pad,), SEG_DUMP, jnp.int32)])
  dst_p = jnp.concatenate([dst, jnp.full((pad,), N, jnp.int32)])

  counts = _sc_counts(seg_p)
  inv = _tc_inv(counts)

  xw1 = _tc_xw(x, W1).reshape(R * N, D)
  agg1 = _sc_scatter(xw1, widx_p, dst_p, seg_p, inv)
  h = _tc_combine(agg1, x, root1, b1, relu=True)

  xw2 = _tc_xw(h, W2).reshape(R * N, D)
  agg2 = _sc_scatter(xw2, widx_p, dst_p, seg_p, inv)
  out = _tc_combine(agg2, h, root2, b2, relu=False)
  return out


# speedup vs baseline: 13.9890x; 1.2861x over previous
import functools
import jax
import jax.numpy as jnp
from jax import lax
from jax.experimental import pallas as pl
from jax.experimental.pallas import tpu as pltpu
from jax.experimental.pallas import tpu_sc as plsc

N = 10000
E = 320000
D = 128
R = 8

NC = 2
NS = 16
NW = NC * NS

B = 128
EPW = 10112
NB = EPW // B
E_PAD = NW * EPW

S_PAD = 81920
SEG_DUMP = N * R
CPS = S_PAD // NS

AGG_N = 10240
APS = AGG_N // NS

_mesh = plsc.VectorSubcoreMesh(core_axis_name="c", subcore_axis_name="s")


def _fill1d(ref, n, val):
  def body(i, carry):
    ref[pl.ds(i * 16, 16)] = jnp.full((16,), val, dtype=ref.dtype)
    return carry
  lax.fori_loop(0, n // 16, body, 0)


def _zero_rows(ref):
  def body(i, carry):
    e = i // (D // 16)
    k = i % (D // 16)
    ref[e, pl.ds(k * 16, 16)] = jnp.zeros((16,), dtype=ref.dtype)
    return carry
  lax.fori_loop(0, B * (D // 16), body, 0)


@functools.partial(
    pl.kernel,
    out_type=jax.ShapeDtypeStruct((NC, S_PAD), jnp.float32),
    mesh=_mesh,
    scratch_types=[
        pltpu.VMEM((B,), jnp.int32),
        pltpu.VMEM((B,), jnp.float32),
        pltpu.VMEM((B,), jnp.float32),
        pltpu.VMEM_SHARED((S_PAD,), jnp.float32),
    ],
    compiler_params=pltpu.CompilerParams(needs_layout_passes=False),
)
def _sc_counts(seg_hbm, cnt_hbm, seg_v, ones_v, zero_v, cnt_sh):
  c = lax.axis_index("c")
  s = lax.axis_index("s")
  w = c * NS + s

  _fill1d(zero_v, B, 0.0)
  _fill1d(ones_v, B, 1.0)

  def zero_body(k, carry):
    pltpu.sync_copy(zero_v, cnt_sh.at[pl.ds(s * CPS + k * B, B)])
    return carry
  lax.fori_loop(0, CPS // B, zero_body, 0)
  plsc.subcore_barrier()

  def acc_body(b, carry):
    pltpu.sync_copy(seg_hbm.at[pl.ds(w * EPW + b * B, B)], seg_v)
    pltpu.sync_copy(ones_v, cnt_sh.at[seg_v], add=True)
    return carry
  lax.fori_loop(0, NB, acc_body, 0)
  plsc.subcore_barrier()

  pltpu.sync_copy(cnt_sh.at[pl.ds(s * CPS, CPS)], cnt_hbm.at[c, pl.ds(s * CPS, CPS)])


@functools.partial(
    pl.kernel,
    out_type=jax.ShapeDtypeStruct((NC, AGG_N, D), jnp.float32),
    mesh=_mesh,
    scratch_types=[
        pltpu.VMEM((2, 3, B), jnp.int32),
        pltpu.VMEM((2, B), jnp.float32),
        pltpu.VMEM((2, B, D), jnp.float32),
        pltpu.SemaphoreType.DMA,
        pltpu.SemaphoreType.DMA,
        pltpu.SemaphoreType.DMA,
        pltpu.SemaphoreType.DMA,
        pltpu.SemaphoreType.DMA,
        pltpu.SemaphoreType.DMA,
        pltpu.VMEM_SHARED((AGG_N, D), jnp.float32),
    ],
    compiler_params=pltpu.CompilerParams(needs_layout_passes=False),
)
def _sc_scatter(xw_hbm, idx_hbm, inv_hbm, agg_hbm,
                idx_v, sc_v, rows_v, g0, g1, s0, s1, w0, w1, agg_sh):
  c = lax.axis_index("c")
  s = lax.axis_index("s")
  w = c * NS + s
  gsem = (g0, g1)
  ssem = (s0, s1)
  wsem = (w0, w1)

  _zero_rows(rows_v.at[0])
  def zero_body(k, carry):
    pltpu.sync_copy(rows_v.at[0], agg_sh.at[pl.ds(s * APS + k * B, B)])
    return carry
  lax.fori_loop(0, APS // B, zero_body, 0)

  plsc.subcore_barrier()

  def scale_rows(p):
    def one_edge(e, carry2):
      spl = plsc.load_gather(sc_v.at[p], [jnp.full((16,), e, jnp.int32)])
      for k in range(D // 16):
        rows_v[p, e, pl.ds(k * 16, 16)] = rows_v[p, e, pl.ds(k * 16, 16)] * spl
      return carry2
    lax.fori_loop(0, B, one_edge, 0, unroll=4)

  def edge_pair(i, carry):
    cps = []
    for p in range(2):
      g = 2 * i + p
      pltpu.sync_copy(idx_hbm.at[w * NB + g], idx_v.at[p])
      rcp = pltpu.async_copy(xw_hbm.at[idx_v.at[p, 0]], rows_v.at[p], gsem[p])
      scp = pltpu.async_copy(inv_hbm.at[idx_v.at[p, 1]], sc_v.at[p], ssem[p])
      cps.append((rcp, scp))
    wcps = []
    for p in range(2):
      rcp, scp = cps[p]
      scp.wait()
      rcp.wait()
      scale_rows(p)
      wcps.append(pltpu.async_copy(
          rows_v.at[p], agg_sh.at[idx_v.at[p, 2]], wsem[p], add=True))
    for p in range(2):
      wcps[p].wait()
    return carry
  lax.fori_loop(0, NB // 2, edge_pair, 0)
  g = NB - 1
  pltpu.sync_copy(idx_hbm.at[w * NB + g], idx_v.at[0])
  rcp = pltpu.async_copy(xw_hbm.at[idx_v.at[0, 0]], rows_v.at[0], gsem[0])
  pltpu.async_copy(inv_hbm.at[idx_v.at[0, 1]], sc_v.at[0], ssem[0]).wait()
  rcp.wait()
  scale_rows(0)
  pltpu.async_copy(rows_v.at[0], agg_sh.at[idx_v.at[0, 2]], wsem[0],
                   add=True).wait()
  plsc.subcore_barrier()

  def dump_body(k, carry):
    pltpu.sync_copy(agg_sh.at[pl.ds(s * APS + k * B, B)],
                    agg_hbm.at[c, pl.ds(s * APS + k * B, B)])
    return carry
  lax.fori_loop(0, APS // B, dump_body, 0)


def _tc_xw_body(x_ref, w_ref, out_ref):
  out_ref[0] = jnp.dot(x_ref[...], w_ref[0],
                       preferred_element_type=jnp.float32)


def _tc_xw(x, W):
  return pl.pallas_call(
      _tc_xw_body,
      grid=(R,),
      in_specs=[
          pl.BlockSpec((N, D), lambda r: (0, 0)),
          pl.BlockSpec((1, D, D), lambda r: (r, 0, 0)),
      ],
      out_specs=pl.BlockSpec((1, N, D), lambda r: (r, 0, 0)),
      out_shape=jax.ShapeDtypeStruct((R, N, D), jnp.float32),
  )(x, W)


def _tc_inv_body(cnt_ref, inv_ref):
  c = cnt_ref[0] + cnt_ref[1]
  inv_ref[...] = 1.0 / jnp.maximum(c, 1.0)


def _tc_inv(counts):
  return pl.pallas_call(
      _tc_inv_body,
      out_shape=jax.ShapeDtypeStruct((S_PAD // 128, 128), jnp.float32),
  )(counts.reshape(2, S_PAD // 128, 128)).reshape(S_PAD)


def _tc_combine_body(agg_ref, x_ref, root_ref, b_ref, out_ref, *, relu):
  acc = agg_ref[0, :N, :] + agg_ref[1, :N, :]
  acc = acc + jnp.dot(x_ref[...], root_ref[...],
                      preferred_element_type=jnp.float32) + b_ref[...]
  out_ref[...] = jnp.maximum(acc, 0.0) if relu else acc


def _tc_combine(agg, x, root, b, relu):
  return pl.pallas_call(
      functools.partial(_tc_combine_body, relu=relu),
      out_shape=jax.ShapeDtypeStruct((N, D), jnp.float32),
  )(agg, x, root, b.reshape(1, D))


def kernel(x, edge_index, edge_type, W1, root1, b1, W2, root2, b2):
  src = edge_index[0].astype(jnp.int32)
  dst = edge_index[1].astype(jnp.int32)
  et = edge_type.astype(jnp.int32)

  pad = E_PAD - E
  widx_p = jnp.concatenate([et * N + src, jnp.zeros((pad,), jnp.int32)])
  seg_p = jnp.concatenate([dst * R + et, jnp.full((pad,), SEG_DUMP, jnp.int32)])
  dst_p = jnp.concatenate([dst, jnp.full((pad,), N, jnp.int32)])
  idx_pack = jnp.stack([widx_p.reshape(-1, B), seg_p.reshape(-1, B),
                        dst_p.reshape(-1, B)], axis=1)

  counts = _sc_counts(seg_p)
  inv = _tc_inv(counts)

  xw1 = _tc_xw(x, W1).reshape(R * N, D)
  agg1 = _sc_scatter(xw1, idx_pack, inv)
  h = _tc_combine(agg1, x, root1, b1, relu=True)

  xw2 = _tc_xw(h, W2).reshape(R * N, D)
  agg2 = _sc_scatter(xw2, idx_pack, inv)
  out = _tc_combine(agg2, h, root2, b2, relu=False)
  return out
